# baseline (speedup 1.0000x reference)
import functools

import jax
import jax.numpy as jnp
from jax import lax
from jax.experimental import pallas as pl
from jax.experimental.pallas import tpu as pltpu
from jax.experimental.pallas import tpu_sc as plsc

N = 10000
E = 320000
D = 128
NC = 2
NS = 16
NW = NC * NS
EPT = E // NW
CHUNK = 80
NBUF = 3
GEDGES = NBUF * CHUNK
EPT_PAD = 10080
NGROUP = EPT_PAD // GEDGES
PAD = EPT_PAD - EPT
NACC = N + 8

RPS_BIG = 632
RPS_SMALL = 624


def _sc_aggregate_body(x_hbm, src_hbm, dst_hbm, out_hbm, sA, dA, sB, dB,
                       rows, acc, gsems, ssems, psems, isem):
    cid = lax.axis_index("c")
    sid = lax.axis_index("s")
    wid = sid * NC + cid

    i_src = pltpu.async_copy(src_hbm.at[wid, 0], sA, psems[0])
    i_dst = pltpu.async_copy(dst_hbm.at[wid, 0], dA, psems[1])

    @pl.when(sid < 2)
    def _():
        st = pl.multiple_of(sid * RPS_BIG, 8)
        pltpu.async_copy(x_hbm.at[pl.ds(st, RPS_BIG)],
                         acc.at[pl.ds(st, RPS_BIG)], isem).wait()

    @pl.when(sid >= 2)
    def _():
        st = pl.multiple_of(2 * RPS_BIG + (sid - 2) * RPS_SMALL, 8)
        pltpu.async_copy(x_hbm.at[pl.ds(st, RPS_SMALL)],
                         acc.at[pl.ds(st, RPS_SMALL)], isem).wait()

    i_src.wait()
    i_dst.wait()
    plsc.subcore_barrier()

    def process(kv, cur_s, cur_d, nxt_s, nxt_d):
        knext = jnp.minimum(kv + 1, NGROUP - 1)
        pf_s = pltpu.async_copy(src_hbm.at[wid, knext], nxt_s, psems[0])
        pf_d = pltpu.async_copy(dst_hbm.at[wid, knext], nxt_d, psems[1])
        gh = []
        for b in range(NBUF):
            gh.append(
                pltpu.async_copy(x_hbm.at[cur_s.at[b]], rows[b], gsems[b])
            )
        sh = []
        for b in range(NBUF):
            gh[b].wait()
            sh.append(
                pltpu.async_copy(rows[b], acc.at[cur_d.at[b]], ssems[b],
                                 add=True)
            )
        pf_s.wait()
        pf_d.wait()
        for b in range(NBUF):
            sh[b].wait()

    @pl.loop(0, NGROUP, step=2)
    def _(k):
        process(k, sA, dA, sB, dB)
        process(k + 1, sB, dB, sA, dA)

    plsc.subcore_barrier()

    @pl.when(sid < 2)
    def _():
        st = pl.multiple_of(sid * RPS_BIG, 8)
        pltpu.async_copy(acc.at[pl.ds(st, RPS_BIG)],
                         out_hbm.at[cid, pl.ds(st, RPS_BIG)], isem).wait()

    @pl.when(sid >= 2)
    def _():
        st = pl.multiple_of(2 * RPS_BIG + (sid - 2) * RPS_SMALL, 8)
        pltpu.async_copy(acc.at[pl.ds(st, RPS_SMALL)],
                         out_hbm.at[cid, pl.ds(st, RPS_SMALL)], isem).wait()


@functools.cache
def _sc_aggregate():
    mesh = plsc.VectorSubcoreMesh(
        core_axis_name="c", subcore_axis_name="s", num_cores=NC, num_subcores=NS
    )

    def wrapper(x_hbm, src_hbm, dst_hbm, out_hbm, sA, dA, sB, dB, *rest):
        rows = list(rest[:NBUF])
        acc = rest[NBUF]
        gsems = list(rest[NBUF + 1:2 * NBUF + 1])
        ssems = list(rest[2 * NBUF + 1:3 * NBUF + 1])
        psems = list(rest[3 * NBUF + 1:3 * NBUF + 3])
        isem = rest[3 * NBUF + 3]
        _sc_aggregate_body(
            x_hbm, src_hbm, dst_hbm, out_hbm, sA, dA, sB, dB,
            rows, acc, gsems, ssems, psems, isem,
        )

    return pl.kernel(
        wrapper,
        out_type=jax.ShapeDtypeStruct((NC, N, D), jnp.float32),
        mesh=mesh,
        scratch_types=(
            [pltpu.VMEM((NBUF, CHUNK), jnp.int32) for _ in range(4)]
            + [pltpu.VMEM((CHUNK, D), jnp.float32) for _ in range(NBUF)]
            + [pltpu.VMEM_SHARED((NACC, D), jnp.float32)]
            + [pltpu.SemaphoreType.DMA for _ in range(2 * NBUF + 2 + 1)]
        ),
    )


def _mlp_body(x_ref, p_ref, w1_ref, b1_ref, g_ref, be_ref, w2_ref, b2_ref,
              wl_ref, bl_ref, o_ref):
    h = p_ref[0] + p_ref[1] - x_ref[...]
    h = jnp.dot(h, w1_ref[...], preferred_element_type=jnp.float32) + b1_ref[...]
    h = jnp.maximum(h, 0.0)
    mean = jnp.mean(h, axis=0, keepdims=True)
    cen = h - mean
    var = jnp.mean(cen * cen, axis=0, keepdims=True)
    h = cen * lax.rsqrt(var + 1e-5) * g_ref[...] + be_ref[...]
    h = jnp.dot(h, w2_ref[...], preferred_element_type=jnp.float32) + b2_ref[...]
    o_ref[...] = (
        jnp.dot(h, wl_ref[...], preferred_element_type=jnp.float32) + bl_ref[...]
    )


_mlp = pl.pallas_call(
    _mlp_body,
    out_shape=jax.ShapeDtypeStruct((N, 10), jnp.float32),
)


def kernel(x, edge_index, W1, b1, gamma, beta, W2, b2, Wlin, blin):
    pad_src = jnp.broadcast_to((jnp.arange(PAD, dtype=jnp.int32) * 125) % N,
                               (NW, PAD))
    pad_dst = jnp.broadcast_to(N + (jnp.arange(PAD, dtype=jnp.int32) % 8),
                               (NW, PAD))
    src = jnp.concatenate([edge_index[0].reshape(NW, EPT), pad_src], axis=1)
    dst = jnp.concatenate([edge_index[1].reshape(NW, EPT), pad_dst], axis=1)
    src = src.reshape(NW, NGROUP, NBUF, CHUNK)
    dst = dst.reshape(NW, NGROUP, NBUF, CHUNK)
    partials = _sc_aggregate()(x, src, dst)
    return _mlp(
        x,
        partials,
        W1,
        b1.reshape(1, -1),
        gamma.reshape(1, -1),
        beta.reshape(1, -1),
        W2,
        b2.reshape(1, -1),
        Wlin,
        blin.reshape(1, -1),
    )


# speedup vs baseline: 1.0731x; 1.0731x over previous
import functools

import jax
import jax.numpy as jnp
from jax import lax
from jax.experimental import pallas as pl
from jax.experimental.pallas import tpu as pltpu
from jax.experimental.pallas import tpu_sc as plsc

N = 10000
E = 320000
D = 128
NC = 2
NS = 16
NW = NC * NS
EPT = E // NW
CHUNK = 40
NBUF = 7
GEDGES = NBUF * CHUNK
EPT_PAD = 10080
NGROUP = EPT_PAD // GEDGES
PAD = EPT_PAD - EPT
NACC = N + 8

RPS_BIG = 632
RPS_SMALL = 624


def _sc_aggregate_body(x_hbm, src_hbm, dst_hbm, out_hbm, sA, dA, sB, dB,
                       rows, acc, gsems, ssems, psems, isem):
    cid = lax.axis_index("c")
    sid = lax.axis_index("s")
    wid = sid * NC + cid

    i_src = pltpu.async_copy(src_hbm.at[wid, 0], sA, psems[0])
    i_dst = pltpu.async_copy(dst_hbm.at[wid, 0], dA, psems[1])

    st = pl.multiple_of(sid * RPS_SMALL, 8)
    seed_main = pltpu.async_copy(x_hbm.at[pl.ds(st, RPS_SMALL)],
                                 acc.at[pl.ds(st, RPS_SMALL)], isem)
    seed_tail = pltpu.async_copy(x_hbm.at[pl.ds(NS * RPS_SMALL, N - NS * RPS_SMALL)],
                                 acc.at[pl.ds(NS * RPS_SMALL, N - NS * RPS_SMALL)],
                                 isem)
    i_src.wait()
    i_dst.wait()

    gh0 = [pltpu.async_copy(x_hbm.at[sA.at[b]], rows[b], gsems[b])
           for b in range(NBUF)]
    seed_main.wait()
    seed_tail.wait()
    plsc.subcore_barrier()

    def process(kv, cur_s, cur_d, nxt_s, nxt_d, gh=None):
        knext = jnp.minimum(kv + 1, NGROUP - 1)
        pf_s = pltpu.async_copy(src_hbm.at[wid, knext], nxt_s, psems[0])
        pf_d = pltpu.async_copy(dst_hbm.at[wid, knext], nxt_d, psems[1])
        if gh is None:
            gh = [pltpu.async_copy(x_hbm.at[cur_s.at[b]], rows[b], gsems[b])
                  for b in range(NBUF)]
        sh = []
        for b in range(NBUF):
            gh[b].wait()
            sh.append(
                pltpu.async_copy(rows[b], acc.at[cur_d.at[b]], ssems[b],
                                 add=True)
            )
        pf_s.wait()
        pf_d.wait()
        for b in range(NBUF):
            sh[b].wait()

    process(0, sA, dA, sB, dB, gh=gh0)
    process(1, sB, dB, sA, dA)

    @pl.loop(2, NGROUP, step=2)
    def _(k):
        process(k, sA, dA, sB, dB)
        process(k + 1, sB, dB, sA, dA)

    plsc.subcore_barrier()

    @pl.when(sid < 2)
    def _():
        st = pl.multiple_of(sid * RPS_BIG, 8)
        pltpu.async_copy(acc.at[pl.ds(st, RPS_BIG)],
                         out_hbm.at[cid, pl.ds(st, RPS_BIG)], isem).wait()

    @pl.when(sid >= 2)
    def _():
        st = pl.multiple_of(2 * RPS_BIG + (sid - 2) * RPS_SMALL, 8)
        pltpu.async_copy(acc.at[pl.ds(st, RPS_SMALL)],
                         out_hbm.at[cid, pl.ds(st, RPS_SMALL)], isem).wait()


@functools.cache
def _sc_aggregate():
    mesh = plsc.VectorSubcoreMesh(
        core_axis_name="c", subcore_axis_name="s", num_cores=NC, num_subcores=NS
    )

    def wrapper(x_hbm, src_hbm, dst_hbm, out_hbm, sA, dA, sB, dB, *rest):
        rows = list(rest[:NBUF])
        acc = rest[NBUF]
        gsems = list(rest[NBUF + 1:2 * NBUF + 1])
        ssems = list(rest[2 * NBUF + 1:3 * NBUF + 1])
        psems = list(rest[3 * NBUF + 1:3 * NBUF + 3])
        isem = rest[3 * NBUF + 3]
        _sc_aggregate_body(
            x_hbm, src_hbm, dst_hbm, out_hbm, sA, dA, sB, dB,
            rows, acc, gsems, ssems, psems, isem,
        )

    return pl.kernel(
        wrapper,
        out_type=jax.ShapeDtypeStruct((NC, N, D), jnp.float32),
        mesh=mesh,
        scratch_types=(
            [pltpu.VMEM((NBUF, CHUNK), jnp.int32) for _ in range(4)]
            + [pltpu.VMEM((CHUNK, D), jnp.float32) for _ in range(NBUF)]
            + [pltpu.VMEM_SHARED((NACC, D), jnp.float32)]
            + [pltpu.SemaphoreType.DMA for _ in range(2 * NBUF + 2 + 1)]
        ),
    )


def _mlp_body(x_ref, p_ref, w1_ref, b1_ref, g_ref, be_ref, w2_ref, b2_ref,
              wl_ref, bl_ref, o_ref):
    h = p_ref[0] + p_ref[1] - x_ref[...]
    h = jnp.dot(h, w1_ref[...], preferred_element_type=jnp.float32) + b1_ref[...]
    h = jnp.maximum(h, 0.0)
    mean = jnp.mean(h, axis=0, keepdims=True)
    cen = h - mean
    var = jnp.mean(cen * cen, axis=0, keepdims=True)
    h = cen * lax.rsqrt(var + 1e-5) * g_ref[...] + be_ref[...]
    h = jnp.dot(h, w2_ref[...], preferred_element_type=jnp.float32) + b2_ref[...]
    o_ref[...] = (
        jnp.dot(h, wl_ref[...], preferred_element_type=jnp.float32) + bl_ref[...]
    )


_mlp = pl.pallas_call(
    _mlp_body,
    out_shape=jax.ShapeDtypeStruct((N, 10), jnp.float32),
)


def kernel(x, edge_index, W1, b1, gamma, beta, W2, b2, Wlin, blin):
    pad_src = jnp.broadcast_to((jnp.arange(PAD, dtype=jnp.int32) * 125) % N,
                               (NW, PAD))
    pad_dst = jnp.broadcast_to(N + (jnp.arange(PAD, dtype=jnp.int32) % 8),
                               (NW, PAD))
    src = jnp.concatenate([edge_index[0].reshape(NW, EPT), pad_src], axis=1)
    dst = jnp.concatenate([edge_index[1].reshape(NW, EPT), pad_dst], axis=1)
    src = src.reshape(NW, NGROUP, NBUF, CHUNK)
    dst = dst.reshape(NW, NGROUP, NBUF, CHUNK)
    partials = _sc_aggregate()(x, src, dst)
    return _mlp(
        x,
        partials,
        W1,
        b1.reshape(1, -1),
        gamma.reshape(1, -1),
        beta.reshape(1, -1),
        W2,
        b2.reshape(1, -1),
        Wlin,
        blin.reshape(1, -1),
    )
